# Initial kernel scaffold; baseline (speedup 1.0000x reference)
import jax
import jax.numpy as jnp
from jax.experimental import pallas as pl


def kernel(x, gru_output, w_omega, u_omega):
    raise NotImplementedError("write your pallas kernel here")


# speedup vs baseline: 1.4253x; 1.4253x over previous
import jax
import jax.numpy as jnp
from jax.experimental import pallas as pl
from jax.experimental.pallas import tpu as pltpu


def _att_body(x_ref, g_ref, w_ref, u_ref, o_ref):
    xb = x_ref[0]
    gb = g_ref[0]
    w = w_ref[...]
    u = u_ref[...]

    t = jnp.tanh(
        jax.lax.dot_general(gb, w, (((1,), (0,)), ((), ())),
                            preferred_element_type=jnp.float32))
    scores = jax.lax.dot_general(u, t, (((1,), (1,)), ((), ())),
                                 preferred_element_type=jnp.float32)
    ones_row = jnp.ones((1, xb.shape[1]), jnp.float32)
    rowsum = jax.lax.dot_general(ones_row, xb, (((1,), (1,)), ((), ())),
                                 preferred_element_type=jnp.float32)

    scores = jnp.where(rowsum == 0.0, jnp.float32(-1e9), scores)
    m = jnp.max(scores, axis=-1, keepdims=True)
    e = jnp.exp(scores - m)
    l = jnp.sum(e, axis=-1, keepdims=True)
    acc = jax.lax.dot_general(e, gb, (((1,), (0,)), ((), ())),
                              preferred_element_type=jnp.float32)
    o_ref[0] = acc / l


def kernel(x, gru_output, w_omega, u_omega):
    B, S, D = x.shape
    A = w_omega.shape[1]
    u2 = u_omega.reshape(1, A)
    return pl.pallas_call(
        _att_body,
        grid=(B,),
        in_specs=[
            pl.BlockSpec((1, S, D), lambda b: (b, 0, 0)),
            pl.BlockSpec((1, S, D), lambda b: (b, 0, 0)),
            pl.BlockSpec((D, A), lambda b: (0, 0)),
            pl.BlockSpec((1, A), lambda b: (0, 0)),
        ],
        out_specs=pl.BlockSpec((1, 1, D), lambda b: (b, 0, 0)),
        out_shape=jax.ShapeDtypeStruct((B, 1, D), jnp.float32),
        compiler_params=pltpu.CompilerParams(
            dimension_semantics=("parallel",),
            vmem_limit_bytes=56 * 1024 * 1024,
        ),
        name="fused_additive_attention",
    )(x, gru_output, w_omega, u2).reshape(B, D)


# speedup vs baseline: 1.4342x; 1.0062x over previous
import jax
import jax.numpy as jnp
from jax.experimental import pallas as pl
from jax.experimental.pallas import tpu as pltpu


def _att_body(x1_ref, x2_ref, g1_ref, g2_ref, w_ref, u_ref, o_ref):
    w = w_ref[...]
    u = u_ref[...]
    halves = ((x1_ref, g1_ref), (x2_ref, g2_ref))

    dn_nt = (((1,), (0,)), ((), ()))
    dn_tb = (((1,), (1,)), ((), ()))

    scores_h = []
    rowsum_h = []
    for x_ref, g_ref in halves:
        xb = x_ref[0]
        gb = g_ref[0]
        t = jnp.tanh(jax.lax.dot_general(
            gb, w, dn_nt, preferred_element_type=jnp.float32))
        scores_h.append(jax.lax.dot_general(
            u, t, dn_tb, preferred_element_type=jnp.float32))
        ones_row = jnp.ones((1, xb.shape[1]), jnp.float32)
        rowsum_h.append(jax.lax.dot_general(
            ones_row, xb, dn_tb, preferred_element_type=jnp.float32))

    scores = jnp.concatenate(scores_h, axis=1)
    rowsum = jnp.concatenate(rowsum_h, axis=1)
    scores = jnp.where(rowsum == 0.0, jnp.float32(-1e9), scores)
    m = jnp.max(scores, axis=-1, keepdims=True)
    e = jnp.exp(scores - m)
    l = jnp.sum(e, axis=-1, keepdims=True)

    sh = e.shape[1] // 2
    acc = jax.lax.dot_general(e[:, :sh], halves[0][1][0], dn_nt,
                              preferred_element_type=jnp.float32)
    acc += jax.lax.dot_general(e[:, sh:], halves[1][1][0], dn_nt,
                               preferred_element_type=jnp.float32)
    o_ref[0] = acc / l


def kernel(x, gru_output, w_omega, u_omega):
    B, S, D = x.shape
    A = w_omega.shape[1]
    u2 = u_omega.reshape(1, A)
    half = pl.BlockSpec((1, S // 2, D), lambda b: (b, 0, 0))
    half2 = pl.BlockSpec((1, S // 2, D), lambda b: (b, 1, 0))
    return pl.pallas_call(
        _att_body,
        grid=(B,),
        in_specs=[
            half, half2,
            pl.BlockSpec((1, S // 2, D), lambda b: (b, 0, 0)),
            pl.BlockSpec((1, S // 2, D), lambda b: (b, 1, 0)),
            pl.BlockSpec((D, A), lambda b: (0, 0)),
            pl.BlockSpec((1, A), lambda b: (0, 0)),
        ],
        out_specs=pl.BlockSpec((1, 1, D), lambda b: (b, 0, 0)),
        out_shape=jax.ShapeDtypeStruct((B, 1, D), jnp.float32),
        compiler_params=pltpu.CompilerParams(
            dimension_semantics=("parallel",),
            vmem_limit_bytes=56 * 1024 * 1024,
        ),
        name="fused_additive_attention",
    )(x, x, gru_output, gru_output, w_omega, u2).reshape(B, D)


# speedup vs baseline: 1.4359x; 1.0012x over previous
import jax
import jax.numpy as jnp
from jax.experimental import pallas as pl
from jax.experimental.pallas import tpu as pltpu


def _att_body(x_ref, g_ref, w_ref, u_ref, o_ref):
    xb = x_ref[0]
    gb = g_ref[0]
    w = w_ref[...]
    u = u_ref[...]

    t = jnp.tanh(
        jax.lax.dot_general(gb, w, (((1,), (0,)), ((), ())),
                            preferred_element_type=jnp.float32))
    scores = jax.lax.dot_general(u, t, (((1,), (1,)), ((), ())),
                                 preferred_element_type=jnp.float32)
    ones_row = jnp.ones((1, xb.shape[1]), jnp.float32)
    rowsum = jax.lax.dot_general(ones_row, xb, (((1,), (1,)), ((), ())),
                                 preferred_element_type=jnp.float32)

    scores = jnp.where(rowsum == 0.0, jnp.float32(-1e9), scores)
    m = jnp.max(scores, axis=-1, keepdims=True)
    e = jnp.exp(scores - m)
    l = jnp.sum(e, axis=-1, keepdims=True)
    acc = jax.lax.dot_general(e, gb, (((1,), (0,)), ((), ())),
                              preferred_element_type=jnp.float32)
    o_ref[0] = acc / l


def kernel(x, gru_output, w_omega, u_omega):
    B, S, D = x.shape
    A = w_omega.shape[1]
    u2 = u_omega.reshape(1, A)
    return pl.pallas_call(
        _att_body,
        grid=(B,),
        in_specs=[
            pl.BlockSpec((1, S, D), lambda b: (b, 0, 0)),
            pl.BlockSpec((1, S, D), lambda b: (b, 0, 0)),
            pl.BlockSpec((D, A), lambda b: (0, 0)),
            pl.BlockSpec((1, A), lambda b: (0, 0)),
        ],
        out_specs=pl.BlockSpec((1, 1, D), lambda b: (b, 0, 0)),
        out_shape=jax.ShapeDtypeStruct((B, 1, D), jnp.float32),
        compiler_params=pltpu.CompilerParams(
            dimension_semantics=("parallel",),
            vmem_limit_bytes=56 * 1024 * 1024,
        ),
        name="fused_additive_attention",
    )(x, gru_output, w_omega, u2).reshape(B, D)
